# baseline (speedup 1.0000x reference)
import jax
import jax.numpy as jnp
from jax.experimental import pallas as pl
from jax.experimental.pallas import tpu as pltpu

_BM = 200


def _body(adj_a_ref, adj_b_ref, ft_all_ref, ft_rows_ref, w1_ref, w2_ref,
          b_ref, out_ref):
    ft_all = ft_all_ref[...]
    w1 = w1_ref[...]
    w2 = w2_ref[...]
    bias = b_ref[...]
    for h, adj_ref in ((0, adj_a_ref), (1, adj_b_ref)):
        neigh = jnp.dot(adj_ref[...], ft_all,
                        preferred_element_type=jnp.float32)
        acc = jnp.dot(neigh, w1, preferred_element_type=jnp.float32)
        acc = acc + jnp.dot(ft_rows_ref[h], w2,
                            preferred_element_type=jnp.float32)
        acc = acc + bias
        out_ref[h] = jnp.maximum(acc, 0.0)


def kernel(ft, adj, W1, W2, b):
    n, in_ch = ft.shape
    out_ch = W1.shape[1]
    bm = _BM
    half = n // 2
    steps = half // bm
    assert half % bm == 0
    b2 = b.reshape(1, out_ch)
    ft_halves = ft.reshape(2, half, in_ch)
    out3 = pl.pallas_call(
        _body,
        grid=(steps,),
        in_specs=[
            pl.BlockSpec((bm, n), lambda i: (i, 0)),
            pl.BlockSpec((bm, n), lambda i, s=steps: (i + s, 0)),
            pl.BlockSpec((n, in_ch), lambda i: (0, 0)),
            pl.BlockSpec((2, bm, in_ch), lambda i: (0, i, 0)),
            pl.BlockSpec((in_ch, out_ch), lambda i: (0, 0)),
            pl.BlockSpec((in_ch, out_ch), lambda i: (0, 0)),
            pl.BlockSpec((1, out_ch), lambda i: (0, 0)),
        ],
        out_specs=pl.BlockSpec((2, bm, out_ch), lambda i: (0, i, 0)),
        out_shape=jax.ShapeDtypeStruct((2, half, out_ch), jnp.float32),
        compiler_params=pltpu.CompilerParams(
            dimension_semantics=("arbitrary",)),
    )(adj, adj, ft, ft_halves, W1, W2, b2)
    return out3.reshape(n, out_ch)


# speedup vs baseline: 1.0682x; 1.0682x over previous
import jax
import jax.numpy as jnp
from jax.experimental import pallas as pl
from jax.experimental.pallas import tpu as pltpu

_BM = 200


def _body(adj_ref, ft_all_ref, w1_ref, w2_ref, b_ref, out_ref):
    i = pl.program_id(0)
    bm = adj_ref.shape[0]
    neigh = jnp.dot(adj_ref[...], ft_all_ref[...],
                    preferred_element_type=jnp.float32)
    acc = jnp.dot(neigh, w1_ref[...], preferred_element_type=jnp.float32)
    ft_rows = ft_all_ref[pl.ds(i * bm, bm), :]
    acc = acc + jnp.dot(ft_rows, w2_ref[...],
                        preferred_element_type=jnp.float32)
    acc = acc + b_ref[...]
    out_ref[...] = jnp.maximum(acc, 0.0)


def kernel(ft, adj, W1, W2, b):
    n, in_ch = ft.shape
    out_ch = W1.shape[1]
    bm = _BM
    assert n % bm == 0
    b2 = b.reshape(1, out_ch)
    return pl.pallas_call(
        _body,
        grid=(n // bm,),
        in_specs=[
            pl.BlockSpec((bm, n), lambda i: (i, 0)),
            pl.BlockSpec((n, in_ch), lambda i: (0, 0)),
            pl.BlockSpec((in_ch, out_ch), lambda i: (0, 0)),
            pl.BlockSpec((in_ch, out_ch), lambda i: (0, 0)),
            pl.BlockSpec((1, out_ch), lambda i: (0, 0)),
        ],
        out_specs=pl.BlockSpec((bm, out_ch), lambda i: (i, 0)),
        out_shape=jax.ShapeDtypeStruct((n, out_ch), jnp.float32),
        compiler_params=pltpu.CompilerParams(
            dimension_semantics=("arbitrary",)),
    )(adj, ft, W1, W2, b2)


# speedup vs baseline: 1.0998x; 1.0295x over previous
import jax
import jax.numpy as jnp
from jax.experimental import pallas as pl
from jax.experimental.pallas import tpu as pltpu

_BM = 400


def _body(adj_ref, ft_all_ref, w1_ref, w2_ref, b_ref, out_ref):
    i = pl.program_id(0)
    bm = adj_ref.shape[0]
    neigh = jnp.dot(adj_ref[...], ft_all_ref[...],
                    preferred_element_type=jnp.float32)
    acc = jnp.dot(neigh, w1_ref[...], preferred_element_type=jnp.float32)
    ft_rows = ft_all_ref[pl.ds(i * bm, bm), :]
    acc = acc + jnp.dot(ft_rows, w2_ref[...],
                        preferred_element_type=jnp.float32)
    acc = acc + b_ref[...]
    out_ref[...] = jnp.maximum(acc, 0.0)


def kernel(ft, adj, W1, W2, b):
    n, in_ch = ft.shape
    out_ch = W1.shape[1]
    bm = _BM
    assert n % bm == 0
    b2 = b.reshape(1, out_ch)
    return pl.pallas_call(
        _body,
        grid=(n // bm,),
        in_specs=[
            pl.BlockSpec((bm, n), lambda i: (i, 0)),
            pl.BlockSpec((n, in_ch), lambda i: (0, 0)),
            pl.BlockSpec((in_ch, out_ch), lambda i: (0, 0)),
            pl.BlockSpec((in_ch, out_ch), lambda i: (0, 0)),
            pl.BlockSpec((1, out_ch), lambda i: (0, 0)),
        ],
        out_specs=pl.BlockSpec((bm, out_ch), lambda i: (i, 0)),
        out_shape=jax.ShapeDtypeStruct((n, out_ch), jnp.float32),
        compiler_params=pltpu.CompilerParams(
            dimension_semantics=("arbitrary",)),
    )(adj, ft, W1, W2, b2)
